# Initial kernel scaffold; baseline (speedup 1.0000x reference)
import jax
import jax.numpy as jnp
from jax.experimental import pallas as pl


def kernel(image, boxes):
    raise NotImplementedError("write your pallas kernel here")


# speedup vs baseline: 1.7309x; 1.7309x over previous
import functools

import jax
import jax.numpy as jnp
from jax import lax
from jax.experimental import pallas as pl
from jax.experimental.pallas import tpu as pltpu
from jax.experimental.pallas import tpu_sc as plsc

ROI_D, ROI_H, ROI_W = 32, 64, 64
B, D, H, W, C = 2, 96, 256, 256, 2
N = 64
WC = W * C
ROW = ROI_W * C
FETCH = ROW + 8
NUM_CROPS = B * N
NUM_WORKERS = 32
CROPS_PER_W = NUM_CROPS // NUM_WORKERS
ZCHUNK = 2
NCHUNK = ROI_D // ZCHUNK
VPR = ROW // 16


def _roi_body(image_hbm, boxes_hbm, out_hbm,
              boxes_v, in0, in1, st0, st1, isem0, isem1, osem0, osem1):
    wid = lax.axis_index("s") * 2 + lax.axis_index("c")

    pltpu.sync_copy(boxes_hbm, boxes_v)

    ins = (in0, in1)
    sts = (st0, st1)
    isems = (isem0, isem1)
    osems = (osem0, osem1)
    T = CROPS_PER_W * NCHUNK

    def crop_of(t):
        j = t // NCHUNK
        idx = wid * CROPS_PER_W + j
        b = idx // N
        n = idx % N
        v = boxes_v[idx]
        z0 = v[0]
        y0 = v[1]
        x0 = v[2]
        xw = pl.multiple_of((x0 // 4) * 8, 8)
        dx2 = x0 * C - xw
        return b, n, z0, y0, xw, dx2

    def in_copy(t, p):
        b, n, z0, y0, xw, _ = crop_of(t)
        ch = t % NCHUNK
        return pltpu.make_async_copy(
            image_hbm.at[b, pl.ds(z0 + ch * ZCHUNK, ZCHUNK),
                         pl.ds(y0, ROI_H), pl.ds(xw, FETCH)],
            ins[p], isems[p])

    def in_wait(p):
        pltpu.make_async_copy(
            image_hbm.at[0, pl.ds(0, ZCHUNK), pl.ds(0, ROI_H),
                         pl.ds(0, FETCH)],
            ins[p], isems[p]).wait()

    def out_wait(p):
        pltpu.make_async_copy(
            sts[p], out_hbm.at[0, 0, pl.ds(0, ZCHUNK)], osems[p]).wait()

    def fix_and_out(t, p):
        b, n, _, _, _, dx2 = crop_of(t)
        ch = t % NCHUNK
        dst_hbm = out_hbm.at[b, n, pl.ds(ch * ZCHUNK, ZCHUNK)]
        src, dst = ins[p], sts[p]

        @pl.when(dx2 == 0)
        def _aligned():
            pltpu.make_async_copy(
                src.at[:, :, pl.ds(0, ROW)], dst_hbm, osems[p]).start()

        @pl.when(dx2 != 0)
        def _shifted():
            @plsc.parallel_loop(0, ZCHUNK * ROI_H, unroll=2)
            def _(r):
                z = r // ROI_H
                y = r % ROI_H
                for col in range(VPR):
                    dst[z, y, pl.ds(col * 16, 16)] = (
                        src[z, y, pl.ds(dx2 + col * 16, 16)])
            pltpu.make_async_copy(dst, dst_hbm, osems[p]).start()

    def pair(g, carry):
        for k in range(2):
            t = 2 * g + k

            @pl.when(t >= 2)
            def _(k=k):
                out_wait(k)

            in_copy(t, k).start()

            @pl.when(t >= 1)
            def _(t=t, k=k):
                in_wait(1 - k)
                fix_and_out(t - 1, 1 - k)
        return carry
    lax.fori_loop(0, T // 2, pair, 0)

    in_wait((T - 1) % 2)
    fix_and_out(T - 1, (T - 1) % 2)
    out_wait((T - 2) % 2)
    out_wait((T - 1) % 2)


def kernel(image, boxes):
    image4 = image.reshape(B, D, H, WC)
    boxes_pad = jnp.pad(boxes.astype(jnp.int32).reshape(NUM_CROPS, 3),
                        ((0, 0), (0, 13)))
    run = functools.partial(
        pl.kernel,
        mesh=plsc.VectorSubcoreMesh(core_axis_name="c", subcore_axis_name="s"),
        compiler_params=pltpu.CompilerParams(use_tc_tiling_on_sc=False),
        out_type=jax.ShapeDtypeStruct((B, N, ROI_D, ROI_H, ROW), jnp.float32),
        scratch_types=[
            pltpu.VMEM((NUM_CROPS, 16), jnp.int32),
            pltpu.VMEM((ZCHUNK, ROI_H, FETCH), jnp.float32),
            pltpu.VMEM((ZCHUNK, ROI_H, FETCH), jnp.float32),
            pltpu.VMEM((ZCHUNK, ROI_H, ROW), jnp.float32),
            pltpu.VMEM((ZCHUNK, ROI_H, ROW), jnp.float32),
            pltpu.SemaphoreType.DMA,
            pltpu.SemaphoreType.DMA,
            pltpu.SemaphoreType.DMA,
            pltpu.SemaphoreType.DMA,
        ],
    )(_roi_body)
    out5 = run(image4, boxes_pad)
    return out5.reshape(B, N, ROI_D, ROI_H, ROI_W, C)
